# baseline (speedup 1.0000x reference)
import jax
import jax.numpy as jnp
from jax import lax
from jax.experimental import pallas as pl
from jax.experimental.pallas import tpu as pltpu
from jax.experimental.pallas import tpu_sc as plsc

B = 16384
NUM = 13
CAT = 26
V = 100000
D = 64

NC = 2
NS = 16
NW = NC * NS

R_TOTAL = CAT * D

SEG = 33408
B1 = SEG
B2 = 2 * SEG
L2 = 33152
T0 = B2 + L2
TL = V - T0


def _sc_body(rbase, rpw, tab_hbm, catx_hbm, out_hbm,
             idx_v, segA, segB, tail_v, obE, obO, rsA, rsB, ts, wsE, wsO):
    npair = rpw // 2
    c = lax.axis_index("c")
    s = lax.axis_index("s")
    wid = s * NC + c
    r0 = rbase + wid * rpw

    def prefetch(row, start, length, buf, sem):
        pltpu.async_copy(tab_hbm.at[row].at[pl.ds(start, length)],
                         buf.at[pl.ds(0, length)], sem)

    def wait_seg(row, start, length, buf, sem):
        pltpu.make_async_copy(tab_hbm.at[row].at[pl.ds(start, length)],
                              buf.at[pl.ds(0, length)], sem).wait()

    def gather_pass(s_id, seg_ref, ob):
        def grp(g, _):
            for u in range(8):
                o = g * 128 + u * 16
                idx16 = idx_v[pl.ds(o, 16)]
                if s_id == 0:
                    m = idx16 < SEG
                    ob[pl.ds(o, 16)] = plsc.load_gather(seg_ref, [idx16],
                                                        mask=m)
                elif s_id == 1:
                    rel = idx16 - B1
                    m = (rel >= 0) & (rel < SEG)
                    v = plsc.load_gather(seg_ref, [rel], mask=m)
                    ob[pl.ds(o, 16)] = jnp.where(m, v, ob[pl.ds(o, 16)])
                else:
                    rel = idx16 - B2
                    m = (rel >= 0) & (rel < L2)
                    v = plsc.load_gather(seg_ref, [rel], mask=m)
                    trel = idx16 - T0
                    tm = trel >= 0
                    tv = plsc.load_gather(tail_v, [trel], mask=tm)
                    cur = jnp.where(m, v, ob[pl.ds(o, 16)])
                    ob[pl.ds(o, 16)] = jnp.where(tm, tv, cur)
            return 0

        lax.fori_loop(0, B // 128, grp, 0)

    def load_idx(row, j_prev):
        j = lax.shift_right_logical(row, 6)

        @pl.when(j != j_prev)
        def _():
            pltpu.sync_copy(catx_hbm.at[j], idx_v)

        return j

    prefetch(r0, 0, SEG, segA, rsA)

    def pair(k, j_prev):
        re = r0 + 2 * k
        ro = re + 1

        @pl.when(k >= 1)
        def _():
            pltpu.make_async_copy(obE, out_hbm.at[re - rbase - 2], wsE).wait()
            pltpu.make_async_copy(obO, out_hbm.at[ro - rbase - 2], wsO).wait()

        j1 = load_idx(re, j_prev)
        prefetch(re, T0, TL, tail_v, ts)
        wait_seg(re, 0, SEG, segA, rsA)
        prefetch(re, B1, SEG, segB, rsB)
        gather_pass(0, segA, obE)
        wait_seg(re, B1, SEG, segB, rsB)
        prefetch(re, B2, L2, segA, rsA)
        gather_pass(1, segB, obE)
        wait_seg(re, B2, L2, segA, rsA)
        prefetch(ro, 0, SEG, segB, rsB)
        wait_seg(re, T0, TL, tail_v, ts)
        gather_pass(2, segA, obE)
        pltpu.async_copy(obE, out_hbm.at[re - rbase], wsE)

        j2 = load_idx(ro, j1)
        prefetch(ro, T0, TL, tail_v, ts)
        wait_seg(ro, 0, SEG, segB, rsB)
        prefetch(ro, B1, SEG, segA, rsA)
        gather_pass(0, segB, obO)
        wait_seg(ro, B1, SEG, segA, rsA)
        prefetch(ro, B2, L2, segB, rsB)
        gather_pass(1, segA, obO)
        wait_seg(ro, B2, L2, segB, rsB)

        @pl.when(k < npair - 1)
        def _():
            prefetch(ro + 1, 0, SEG, segA, rsA)

        wait_seg(ro, T0, TL, tail_v, ts)
        gather_pass(2, segB, obO)
        pltpu.async_copy(obO, out_hbm.at[ro - rbase], wsO)
        return j2

    lax.fori_loop(0, npair, pair, jnp.int32(-1))

    pltpu.make_async_copy(obE, out_hbm.at[r0 - rbase + rpw - 2], wsE).wait()
    pltpu.make_async_copy(obO, out_hbm.at[r0 - rbase + rpw - 1], wsO).wait()


def _sc_gather(rbase, rc, tab_t, cat_x_t):
    rpw = rc // NW
    mesh = plsc.VectorSubcoreMesh(core_axis_name="c", subcore_axis_name="s")
    return pl.kernel(
        lambda *refs: _sc_body(rbase, rpw, *refs),
        out_type=jax.ShapeDtypeStruct((rc, B), jnp.float32),
        mesh=mesh,
        scratch_types=[
            pltpu.VMEM((B,), jnp.int32),
            pltpu.VMEM((SEG,), jnp.float32),
            pltpu.VMEM((SEG,), jnp.float32),
            pltpu.VMEM((TL,), jnp.float32),
            pltpu.VMEM((B,), jnp.float32),
            pltpu.VMEM((B,), jnp.float32),
            pltpu.SemaphoreType.DMA,
            pltpu.SemaphoreType.DMA,
            pltpu.SemaphoreType.DMA,
            pltpu.SemaphoreType.DMA,
            pltpu.SemaphoreType.DMA,
        ],
        compiler_params=pltpu.CompilerParams(use_tc_tiling_on_sc=True,
                                             needs_layout_passes=False),
    )(tab_t, cat_x_t)


def _tc1_body(numx_ref, staged_ref, wn_ref, bn_ref, nw_ref, cw_ref, fp_ref,
              g_ref, be_ref, out_ref):
    x = numx_ref[...]
    wn = wn_ref[...] * nw_ref[...]
    bn = bn_ref[...] * nw_ref[...]
    ntok = wn[:, :, None] * x[:, None, :] + bn[:, :, None]
    ctok = staged_ref[...].reshape(F1, D, -1) * cw_ref[...][:, :, None]
    tok = jnp.concatenate([ntok, ctok], axis=0) + fp_ref[...][:, :, None]
    mean = jnp.mean(tok, axis=1, keepdims=True)
    cen = tok - mean
    var = jnp.mean(cen * cen, axis=1, keepdims=True)
    y = cen * lax.rsqrt(var + 1e-5)
    out_ref[...] = (y * g_ref[...][None, :, None]
                    + be_ref[...][None, :, None])


def _tc_cat_body(part_ref, staged_ref, cw_ref, fp_ref, g_ref, be_ref,
                 out_ref):
    del part_ref
    f = pl.program_id(0)
    tok = staged_ref[0] * cw_ref[f, 0] + fp_ref[f][:, None]
    mean = jnp.mean(tok, axis=0, keepdims=True)
    cen = tok - mean
    var = jnp.mean(cen * cen, axis=0, keepdims=True)
    y = cen * lax.rsqrt(var + 1e-5)
    out_ref[0] = y * g_ref[...][:, None] + be_ref[...][:, None]


TB = 512
NF = NUM + CAT
F1 = 13
F2 = 8
F3 = CAT - F1 - F2


@jax.jit
def _fused(tab_t, cat_x_t, num_x_t, W_num, b_num, num_w, cat_w, feat_pos,
           gamma, beta):
    staged0 = _sc_gather(0, F1 * D, tab_t, cat_x_t)
    staged1 = _sc_gather(F1 * D, F2 * D, tab_t, cat_x_t)
    staged2 = _sc_gather((F1 + F2) * D, F3 * D, tab_t, cat_x_t)
    grid = (B // TB,)

    part = pl.pallas_call(
        _tc1_body,
        grid=grid,
        in_specs=[
            pl.BlockSpec((NUM, TB), lambda i: (0, i)),
            pl.BlockSpec((F1 * D, TB), lambda i: (0, i)),
            pl.BlockSpec((NUM, D), lambda i: (0, 0)),
            pl.BlockSpec((NUM, D), lambda i: (0, 0)),
            pl.BlockSpec((NUM, 1), lambda i: (0, 0)),
            pl.BlockSpec((F1, 1), lambda i: (0, 0)),
            pl.BlockSpec((NUM + F1, D), lambda i: (0, 0)),
            pl.BlockSpec((D,), lambda i: (0,)),
            pl.BlockSpec((D,), lambda i: (0,)),
        ],
        out_specs=pl.BlockSpec((NUM + F1, D, TB), lambda i: (0, 0, i)),
        out_shape=jax.ShapeDtypeStruct((NF, D, B), jnp.float32),
    )(num_x_t, staged0, W_num, b_num, num_w, cat_w[:F1],
      feat_pos[:NUM + F1], gamma, beta)

    def tc_cat(prev, staged, f0, nf):
        return pl.pallas_call(
            _tc_cat_body,
            grid=(nf, B // TB),
            in_specs=[
                pl.BlockSpec(memory_space=pl.ANY),
                pl.BlockSpec((1, D, TB), lambda f, i: (f, 0, i)),
                pl.BlockSpec((nf, 1), lambda f, i: (0, 0)),
                pl.BlockSpec((nf, D), lambda f, i: (0, 0)),
                pl.BlockSpec((D,), lambda f, i: (0,)),
                pl.BlockSpec((D,), lambda f, i: (0,)),
            ],
            out_specs=pl.BlockSpec((1, D, TB),
                                   lambda f, i, f0=f0: (NUM + f0 + f, 0, i)),
            out_shape=jax.ShapeDtypeStruct((NF, D, B), jnp.float32),
            input_output_aliases={0: 0},
        )(prev, staged.reshape(nf, D, B), cat_w[f0:f0 + nf],
          feat_pos[NUM + f0:NUM + f0 + nf], gamma, beta)

    part = tc_cat(part, staged1, F1, F2)
    out_t = tc_cat(part, staged2, F1 + F2, F3)
    return jnp.transpose(out_t, (2, 0, 1))


def kernel(num_x, cat_x, W_num, b_num, num_w, cat_tables, cat_w, feat_pos,
           gamma, beta):
    tab_t = jnp.transpose(cat_tables, (0, 2, 1)).reshape(R_TOTAL, V)
    cat_x_t = jnp.transpose(cat_x, (1, 0))
    num_x_t = jnp.transpose(num_x, (1, 0))
    return _fused(tab_t, cat_x_t, num_x_t, W_num, b_num, num_w, cat_w,
                  feat_pos, gamma, beta)


# speedup vs baseline: 1.0905x; 1.0905x over previous
import jax
import jax.numpy as jnp
from jax import lax
from jax.experimental import pallas as pl
from jax.experimental.pallas import tpu as pltpu
from jax.experimental.pallas import tpu_sc as plsc

B = 16384
NUM = 13
CAT = 26
V = 100000
D = 64

NC = 2
NS = 16
NW = NC * NS

R_TOTAL = CAT * D

SEG = 33408
B1 = SEG
B2 = 2 * SEG
L2 = 33152
T0 = B2 + L2
TL = V - T0


def _sc_body(rbase, rpw, tab_hbm, catx_hbm, out_hbm,
             idx_v, segA, segB, tail_v, obE, obO, rsA, rsB, ts, wsE, wsO):
    npair = rpw // 2
    c = lax.axis_index("c")
    s = lax.axis_index("s")
    wid = s * NC + c
    r0 = rbase + wid * rpw

    def prefetch(row, start, length, buf, sem):
        pltpu.async_copy(tab_hbm.at[row].at[pl.ds(start, length)],
                         buf.at[pl.ds(0, length)], sem)

    def wait_seg(row, start, length, buf, sem):
        pltpu.make_async_copy(tab_hbm.at[row].at[pl.ds(start, length)],
                              buf.at[pl.ds(0, length)], sem).wait()

    def gather_pass(s_id, seg_ref, ob):
        def grp(g, _):
            for u in range(8):
                o = g * 128 + u * 16
                idx16 = idx_v[pl.ds(o, 16)]
                if s_id == 0:
                    m = idx16 < SEG
                    ob[pl.ds(o, 16)] = plsc.load_gather(seg_ref, [idx16],
                                                        mask=m)
                elif s_id == 1:
                    rel = idx16 - B1
                    m = (rel >= 0) & (rel < SEG)
                    v = plsc.load_gather(seg_ref, [rel], mask=m)
                    ob[pl.ds(o, 16)] = jnp.where(m, v, ob[pl.ds(o, 16)])
                else:
                    rel = idx16 - B2
                    m = (rel >= 0) & (rel < L2)
                    v = plsc.load_gather(seg_ref, [rel], mask=m)
                    trel = idx16 - T0
                    tm = trel >= 0
                    tv = plsc.load_gather(tail_v, [trel], mask=tm)
                    cur = jnp.where(m, v, ob[pl.ds(o, 16)])
                    ob[pl.ds(o, 16)] = jnp.where(tm, tv, cur)
            return 0

        lax.fori_loop(0, B // 128, grp, 0)

    def load_idx(row, j_prev):
        j = lax.shift_right_logical(row, 6)

        @pl.when(j != j_prev)
        def _():
            pltpu.sync_copy(catx_hbm.at[j], idx_v)

        return j

    prefetch(r0, 0, SEG, segA, rsA)

    def pair(k, j_prev):
        re = r0 + 2 * k
        ro = re + 1

        @pl.when(k >= 1)
        def _():
            pltpu.make_async_copy(obE, out_hbm.at[re - rbase - 2], wsE).wait()
            pltpu.make_async_copy(obO, out_hbm.at[ro - rbase - 2], wsO).wait()

        j1 = load_idx(re, j_prev)
        prefetch(re, T0, TL, tail_v, ts)
        wait_seg(re, 0, SEG, segA, rsA)
        prefetch(re, B1, SEG, segB, rsB)
        gather_pass(0, segA, obE)
        wait_seg(re, B1, SEG, segB, rsB)
        prefetch(re, B2, L2, segA, rsA)
        gather_pass(1, segB, obE)
        wait_seg(re, B2, L2, segA, rsA)
        prefetch(ro, 0, SEG, segB, rsB)
        wait_seg(re, T0, TL, tail_v, ts)
        gather_pass(2, segA, obE)
        pltpu.async_copy(obE, out_hbm.at[re - rbase], wsE)

        j2 = load_idx(ro, j1)
        prefetch(ro, T0, TL, tail_v, ts)
        wait_seg(ro, 0, SEG, segB, rsB)
        prefetch(ro, B1, SEG, segA, rsA)
        gather_pass(0, segB, obO)
        wait_seg(ro, B1, SEG, segA, rsA)
        prefetch(ro, B2, L2, segB, rsB)
        gather_pass(1, segA, obO)
        wait_seg(ro, B2, L2, segB, rsB)

        @pl.when(k < npair - 1)
        def _():
            prefetch(ro + 1, 0, SEG, segA, rsA)

        wait_seg(ro, T0, TL, tail_v, ts)
        gather_pass(2, segB, obO)
        pltpu.async_copy(obO, out_hbm.at[ro - rbase], wsO)
        return j2

    lax.fori_loop(0, npair, pair, jnp.int32(-1))

    pltpu.make_async_copy(obE, out_hbm.at[r0 - rbase + rpw - 2], wsE).wait()
    pltpu.make_async_copy(obO, out_hbm.at[r0 - rbase + rpw - 1], wsO).wait()


def _sc_gather(rbase, rc, tab_t, cat_x_t):
    rpw = rc // NW
    mesh = plsc.VectorSubcoreMesh(core_axis_name="c", subcore_axis_name="s")
    return pl.kernel(
        lambda *refs: _sc_body(rbase, rpw, *refs),
        out_type=jax.ShapeDtypeStruct((rc, B), jnp.float32),
        mesh=mesh,
        scratch_types=[
            pltpu.VMEM((B,), jnp.int32),
            pltpu.VMEM((SEG,), jnp.float32),
            pltpu.VMEM((SEG,), jnp.float32),
            pltpu.VMEM((TL,), jnp.float32),
            pltpu.VMEM((B,), jnp.float32),
            pltpu.VMEM((B,), jnp.float32),
            pltpu.SemaphoreType.DMA,
            pltpu.SemaphoreType.DMA,
            pltpu.SemaphoreType.DMA,
            pltpu.SemaphoreType.DMA,
            pltpu.SemaphoreType.DMA,
        ],
        compiler_params=pltpu.CompilerParams(use_tc_tiling_on_sc=True,
                                             needs_layout_passes=False),
    )(tab_t, cat_x_t)


def _tc1_body(numx_ref, staged_ref, wn_ref, bn_ref, nw_ref, cw_ref, fp_ref,
              g_ref, be_ref, out_ref):
    x = numx_ref[...]
    wn = wn_ref[...] * nw_ref[...]
    bn = bn_ref[...] * nw_ref[...]
    ntok = wn[:, :, None] * x[:, None, :] + bn[:, :, None]
    ctok = staged_ref[...].reshape(F1, D, -1) * cw_ref[...][:, :, None]
    tok = jnp.concatenate([ntok, ctok], axis=0) + fp_ref[...][:, :, None]
    mean = jnp.mean(tok, axis=1, keepdims=True)
    cen = tok - mean
    var = jnp.mean(cen * cen, axis=1, keepdims=True)
    y = cen * lax.rsqrt(var + 1e-5)
    out_ref[...] = (y * g_ref[...][None, :, None]
                    + be_ref[...][None, :, None])


def _tc2_body(part_ref, staged_ref, cw_ref, fp_ref, g_ref, be_ref,
              out_ref):
    del part_ref
    tok = (staged_ref[...].reshape(F2, D, -1) * cw_ref[...][:, :, None]
           + fp_ref[...][:, :, None])
    mean = jnp.mean(tok, axis=1, keepdims=True)
    cen = tok - mean
    var = jnp.mean(cen * cen, axis=1, keepdims=True)
    y = cen * lax.rsqrt(var + 1e-5)
    out_ref[...] = (y * g_ref[...][None, :, None]
                    + be_ref[...][None, :, None])


TB = 512
NF = NUM + CAT
F1 = 23
F2 = CAT - F1


@jax.jit
def _fused(tab_t, cat_x_t, num_x_t, W_num, b_num, num_w, cat_w, feat_pos,
           gamma, beta):
    staged0 = _sc_gather(0, F1 * D, tab_t, cat_x_t)
    staged1 = _sc_gather(F1 * D, F2 * D, tab_t, cat_x_t)
    grid = (B // TB,)

    part = pl.pallas_call(
        _tc1_body,
        grid=grid,
        in_specs=[
            pl.BlockSpec((NUM, TB), lambda i: (0, i)),
            pl.BlockSpec((F1 * D, TB), lambda i: (0, i)),
            pl.BlockSpec((NUM, D), lambda i: (0, 0)),
            pl.BlockSpec((NUM, D), lambda i: (0, 0)),
            pl.BlockSpec((NUM, 1), lambda i: (0, 0)),
            pl.BlockSpec((F1, 1), lambda i: (0, 0)),
            pl.BlockSpec((NUM + F1, D), lambda i: (0, 0)),
            pl.BlockSpec((D,), lambda i: (0,)),
            pl.BlockSpec((D,), lambda i: (0,)),
        ],
        out_specs=pl.BlockSpec((NUM + F1, D, TB), lambda i: (0, 0, i)),
        out_shape=jax.ShapeDtypeStruct((NF, D, B), jnp.float32),
    )(num_x_t, staged0, W_num, b_num, num_w, cat_w[:F1],
      feat_pos[:NUM + F1], gamma, beta)

    out_t = pl.pallas_call(
        _tc2_body,
        grid=grid,
        in_specs=[
            pl.BlockSpec(memory_space=pl.ANY),
            pl.BlockSpec((F2 * D, TB), lambda i: (0, i)),
            pl.BlockSpec((F2, 1), lambda i: (0, 0)),
            pl.BlockSpec((F2, D), lambda i: (0, 0)),
            pl.BlockSpec((D,), lambda i: (0,)),
            pl.BlockSpec((D,), lambda i: (0,)),
        ],
        out_specs=pl.BlockSpec((F2, D, TB),
                               lambda i: ((NUM + F1) // F2, 0, i)),
        out_shape=jax.ShapeDtypeStruct((NF, D, B), jnp.float32),
        input_output_aliases={0: 0},
    )(part, staged1, cat_w[F1:], feat_pos[NUM + F1:], gamma, beta)
    return jnp.transpose(out_t, (2, 0, 1))


def kernel(num_x, cat_x, W_num, b_num, num_w, cat_tables, cat_w, feat_pos,
           gamma, beta):
    tab_t = jnp.transpose(cat_tables, (0, 2, 1)).reshape(R_TOTAL, V)
    cat_x_t = jnp.transpose(cat_x, (1, 0))
    num_x_t = jnp.transpose(num_x, (1, 0))
    return _fused(tab_t, cat_x_t, num_x_t, W_num, b_num, num_w, cat_w,
                  feat_pos, gamma, beta)
